# Initial kernel scaffold; baseline (speedup 1.0000x reference)
import jax
import jax.numpy as jnp
from jax.experimental import pallas as pl


def kernel(w_seq, c_seq, char_embedding, word_embedding, w_ih1, w_hh1, b_ih1, b_hh1, c_hx, c_cx, w_ih2, w_hh2, b_ih2, b_hh2, hx, cx, w_out, b_out):
    raise NotImplementedError("write your pallas kernel here")


# speedup vs baseline: 1.1413x; 1.1413x over previous
import jax
import jax.numpy as jnp
from jax.experimental import pallas as pl
from jax.experimental.pallas import tpu as pltpu


def _sigmoid(x):
    return 0.5 * (jnp.tanh(0.5 * x) + 1.0)


def _full_spec(shape):
    nd = len(shape)
    return pl.BlockSpec(shape, lambda i, _nd=nd: (0,) * _nd)


def kernel(w_seq, c_seq, char_embedding, word_embedding,
           w_ih1, w_hh1, b_ih1, b_hh1, c_hx, c_cx,
           w_ih2, w_hh2, b_ih2, b_hh2, hx, cx, w_out, b_out):
    W = int(w_seq.shape[0])
    Lc = int(c_seq.shape[1])
    Hc = int(c_hx.shape[-1])
    H = int(hx.shape[-1])
    T = int(w_out.shape[0])
    Cs = int(char_embedding.shape[0])
    Wp = ((W + 7) // 8) * 8
    bf = jnp.bfloat16

    cseq = c_seq.T
    if Wp != W:
        cseq = jnp.pad(cseq, ((0, 0), (0, Wp - W)))
    cseq2d = cseq.reshape(Lc * Wp, 1)

    wemb = jnp.take(word_embedding, w_seq, axis=0).astype(bf)
    if Wp != W:
        wemb = jnp.pad(wemb, ((0, Wp - W), (0, 0)))

    ce = char_embedding.astype(bf)
    w_ih1t = w_ih1.T.astype(bf)
    w_hh1t = w_hh1.T.astype(bf)
    b1 = (b_ih1 + b_hh1)[None, :]
    w2ct = w_ih2[:, :Hc].T.astype(bf)
    w2wt = w_ih2[:, Hc:].T.astype(bf)
    w_hh2t = w_hh2.T.astype(bf)
    b2 = (b_ih2 + b_hh2)[None, :]
    w_outt = w_out.T.astype(bf)
    bout = b_out[None, :]
    ch0 = c_hx.reshape(1, Hc)
    cc0 = c_cx.reshape(1, Hc)
    h0 = hx.reshape(1, H)
    c0 = cx.reshape(1, H)

    def body(cseq_ref, wemb_ref, ce_ref, wih1t_ref, whh1t_ref, b1_ref,
             ch0_ref, cc0_ref, w2ct_ref, w2wt_ref, whh2t_ref, b2_ref,
             h0_ref, c0_ref, woutt_ref, bout_ref,
             out_ref, xg1_scr, xg2_scr, ho_scr):
        ce_proj = jnp.dot(ce_ref[...], wih1t_ref[...],
                          preferred_element_type=jnp.float32).astype(bf)
        idx = cseq_ref[...]
        iota = jax.lax.broadcasted_iota(jnp.int32, (Lc * Wp, Cs), 1)
        onehot = (iota == idx).astype(bf)
        xg1_scr[...] = (jnp.dot(onehot, ce_proj,
                                preferred_element_type=jnp.float32)
                        + b1_ref[...])

        whh1t = whh1t_ref[...]
        h = jnp.broadcast_to(ch0_ref[...], (Wp, Hc))
        c = jnp.broadcast_to(cc0_ref[...], (Wp, Hc))
        for t in range(Lc):
            gates = xg1_scr[t * Wp:(t + 1) * Wp, :] + jnp.dot(
                h.astype(bf), whh1t, preferred_element_type=jnp.float32)
            sif = _sigmoid(gates[:, :2 * Hc])
            g = jnp.tanh(gates[:, 2 * Hc:3 * Hc])
            so = _sigmoid(gates[:, 3 * Hc:])
            c = sif[:, Hc:] * c + sif[:, :Hc] * g
            h = so * jnp.tanh(c)

        xg2_scr[...] = (jnp.dot(h.astype(bf), w2ct_ref[...],
                                preferred_element_type=jnp.float32)
                        + jnp.dot(wemb_ref[...], w2wt_ref[...],
                                  preferred_element_type=jnp.float32)
                        + b2_ref[...])

        whh2t = whh2t_ref[...]
        h2 = h0_ref[...]
        c2 = c0_ref[...]
        for w in range(W):
            gates = xg2_scr[w:w + 1, :] + jnp.dot(
                h2.astype(bf), whh2t, preferred_element_type=jnp.float32)
            sif = _sigmoid(gates[:, :2 * H])
            g = jnp.tanh(gates[:, 2 * H:3 * H])
            so = _sigmoid(gates[:, 3 * H:])
            c2 = sif[:, H:] * c2 + sif[:, :H] * g
            h2 = so * jnp.tanh(c2)
            ho_scr[w:w + 1, :] = h2.astype(bf)

        tag = (jnp.dot(ho_scr[...], woutt_ref[...],
                       preferred_element_type=jnp.float32) + bout_ref[...])
        m = jnp.max(tag, axis=1, keepdims=True)
        z = tag - m
        lse = jnp.log(jnp.sum(jnp.exp(z), axis=1, keepdims=True))
        out_ref[...] = z - lse

    inputs = (cseq2d, wemb, ce, w_ih1t, w_hh1t, b1, ch0, cc0,
              w2ct, w2wt, w_hh2t, b2, h0, c0, w_outt, bout)

    out = pl.pallas_call(
        body,
        out_shape=jax.ShapeDtypeStruct((W, T), jnp.float32),
        grid=(1,),
        in_specs=[_full_spec(x.shape) for x in inputs],
        out_specs=_full_spec((W, T)),
        scratch_shapes=[pltpu.VMEM((Lc * Wp, 4 * Hc), jnp.float32),
                        pltpu.VMEM((Wp, 4 * H), jnp.float32),
                        pltpu.VMEM((W, H), bf)],
        compiler_params=pltpu.CompilerParams(
            dimension_semantics=("arbitrary",)),
    )(*inputs)
    return out


# speedup vs baseline: 2.6088x; 2.2858x over previous
import jax
import jax.numpy as jnp
from jax.experimental import pallas as pl
from jax.experimental.pallas import tpu as pltpu


def _sigmoid(x):
    return 0.5 * (jnp.tanh(0.5 * x) + 1.0)


def kernel(w_seq, c_seq, char_embedding, word_embedding,
           w_ih1, w_hh1, b_ih1, b_hh1, c_hx, c_cx,
           w_ih2, w_hh2, b_ih2, b_hh2, hx, cx, w_out, b_out):
    W = int(w_seq.shape[0])
    Lc = int(c_seq.shape[1])
    Hc = int(c_hx.shape[-1])
    H = int(hx.shape[-1])
    T = int(w_out.shape[0])
    Cs = int(char_embedding.shape[0])
    Wp = ((W + 7) // 8) * 8
    bf = jnp.bfloat16

    cseq = c_seq
    if Wp != W:
        cseq = jnp.pad(cseq, ((0, Wp - W), (0, 0)))

    def body(wseq_ref, cseq_ref, wemb_hbm, ce_ref,
             wih1_ref, whh1_ref, bih1_ref, bhh1_ref, ch0_ref, cc0_ref,
             wih2_ref, whh2_ref, bih2_ref, bhh2_ref, h0_ref, c0_ref,
             wout_ref, bout_ref,
             out_ref, wemb_scr, xg2_scr, ho_scr, dma_sem):
        for w in range(W):
            pltpu.make_async_copy(wemb_hbm.at[wseq_ref[w]],
                                  wemb_scr.at[w], dma_sem).start()

        wih1t = wih1_ref[...].T.astype(bf)
        whh1t = whh1_ref[...].T.astype(bf)
        b1 = bih1_ref[...] + bhh1_ref[...]
        ce_projb = (jnp.dot(ce_ref[...].astype(bf), wih1t,
                            preferred_element_type=jnp.float32)
                    + b1).astype(bf)

        h = jnp.broadcast_to(ch0_ref[...], (Wp, Hc))
        c = jnp.broadcast_to(cc0_ref[...], (Wp, Hc))
        iota = jax.lax.broadcasted_iota(jnp.int32, (Wp, Cs), 1)
        for t in range(Lc):
            onehot = (iota == cseq_ref[:, t:t + 1]).astype(bf)
            gates = (jnp.dot(onehot, ce_projb,
                             preferred_element_type=jnp.float32)
                     + jnp.dot(h.astype(bf), whh1t,
                               preferred_element_type=jnp.float32))
            sif = _sigmoid(gates[:, :2 * Hc])
            g = jnp.tanh(gates[:, 2 * Hc:3 * Hc])
            so = _sigmoid(gates[:, 3 * Hc:])
            c = sif[:, Hc:] * c + sif[:, :Hc] * g
            h = so * jnp.tanh(c)

        w2ct = wih2_ref[:, :Hc].T.astype(bf)
        w2wt = wih2_ref[:, Hc:].T.astype(bf)
        b2 = bih2_ref[...] + bhh2_ref[...]
        pltpu.make_async_copy(wemb_scr.at[pl.ds(0, W)],
                              wemb_scr.at[pl.ds(0, W)], dma_sem).wait()
        xg2_scr[...] = (jnp.dot(h.astype(bf), w2ct,
                                preferred_element_type=jnp.float32)
                        + jnp.dot(wemb_scr[...].astype(bf), w2wt,
                                  preferred_element_type=jnp.float32)
                        + b2)

        whh2t = whh2_ref[...].T.astype(bf)
        h2 = h0_ref[...]
        c2 = c0_ref[...]
        for w in range(W):
            gates = xg2_scr[w:w + 1, :] + jnp.dot(
                h2.astype(bf), whh2t, preferred_element_type=jnp.float32)
            sif = _sigmoid(gates[:, :2 * H])
            g = jnp.tanh(gates[:, 2 * H:3 * H])
            so = _sigmoid(gates[:, 3 * H:])
            c2 = sif[:, H:] * c2 + sif[:, :H] * g
            h2 = so * jnp.tanh(c2)
            ho_scr[w:w + 1, :] = h2.astype(bf)

        woutt = wout_ref[...].T.astype(bf)
        tag = (jnp.dot(ho_scr[...], woutt,
                       preferred_element_type=jnp.float32) + bout_ref[...])
        m = jnp.max(tag, axis=1, keepdims=True)
        z = tag - m
        lse = jnp.log(jnp.sum(jnp.exp(z), axis=1, keepdims=True))
        out_ref[...] = z - lse

    vmem_inputs = (char_embedding,
                   w_ih1, w_hh1, b_ih1.reshape(1, -1), b_hh1.reshape(1, -1),
                   c_hx.reshape(1, Hc), c_cx.reshape(1, Hc),
                   w_ih2, w_hh2, b_ih2.reshape(1, -1), b_hh2.reshape(1, -1),
                   hx.reshape(1, H), cx.reshape(1, H),
                   w_out, b_out.reshape(1, -1))

    def _full(shape):
        nd = len(shape)
        return pl.BlockSpec(shape, lambda i, _nd=nd: (0,) * _nd)

    out = pl.pallas_call(
        body,
        out_shape=jax.ShapeDtypeStruct((Wp, T), jnp.float32),
        grid=(1,),
        in_specs=[pl.BlockSpec(memory_space=pltpu.SMEM),
                  _full(cseq.shape),
                  pl.BlockSpec(memory_space=pl.ANY)]
                 + [_full(x.shape) for x in vmem_inputs],
        out_specs=_full((Wp, T)),
        scratch_shapes=[pltpu.VMEM((Wp, char_embedding.shape[1]), jnp.float32),
                        pltpu.VMEM((Wp, 4 * H), jnp.float32),
                        pltpu.VMEM((Wp, H), bf),
                        pltpu.SemaphoreType.DMA],
        compiler_params=pltpu.CompilerParams(
            dimension_semantics=("arbitrary",)),
    )(w_seq, cseq, word_embedding, *vmem_inputs)
    if Wp != W:
        out = out[:W]
    return out


# speedup vs baseline: 2.6699x; 1.0234x over previous
import jax
import jax.numpy as jnp
from jax.experimental import pallas as pl
from jax.experimental.pallas import tpu as pltpu


def _sigmoid(x):
    return 0.5 * (jnp.tanh(0.5 * x) + 1.0)


def kernel(w_seq, c_seq, char_embedding, word_embedding,
           w_ih1, w_hh1, b_ih1, b_hh1, c_hx, c_cx,
           w_ih2, w_hh2, b_ih2, b_hh2, hx, cx, w_out, b_out):
    W = int(w_seq.shape[0])
    Lc = int(c_seq.shape[1])
    Hc = int(c_hx.shape[-1])
    Ew = int(word_embedding.shape[1])
    H = int(hx.shape[-1])
    T = int(w_out.shape[0])
    Cs = int(char_embedding.shape[0])
    Wp = ((W + 7) // 8) * 8
    bf = jnp.bfloat16

    cseq = c_seq
    if Wp != W:
        cseq = jnp.pad(cseq, ((0, Wp - W), (0, 0)))

    def body(wseq_ref, cseq_ref, wemb_hbm, wih2_hbm, whh2_hbm, wout_hbm,
             ce_ref, wih1_ref, whh1_ref, bih1_ref, bhh1_ref,
             ch0_ref, cc0_ref, bih2_ref, bhh2_ref, h0_ref, c0_ref, bout_ref,
             out_ref,
             wemb_scr, wih2_scr, whh2_scr, wout_scr, xg2_scr, ho_scr,
             gsem, wsem):
        for w in range(W):
            pltpu.make_async_copy(wemb_hbm.at[wseq_ref[w]],
                                  wemb_scr.at[w], gsem).start()
        pltpu.make_async_copy(wih2_hbm, wih2_scr, wsem).start()
        pltpu.make_async_copy(whh2_hbm, whh2_scr, wsem).start()
        pltpu.make_async_copy(wout_hbm, wout_scr, wsem).start()

        wih1t = wih1_ref[...].T.astype(bf)
        whh1t = whh1_ref[...].T.astype(bf)
        b1 = bih1_ref[...] + bhh1_ref[...]
        ce_projb = (jnp.dot(ce_ref[...].astype(bf), wih1t,
                            preferred_element_type=jnp.float32)
                    + b1).astype(bf)

        h = jnp.broadcast_to(ch0_ref[...], (Wp, Hc))
        c = jnp.broadcast_to(cc0_ref[...], (Wp, Hc))
        iota = jax.lax.broadcasted_iota(jnp.int32, (Wp, Cs), 1)
        for t in range(Lc):
            onehot = (iota == cseq_ref[:, t:t + 1]).astype(bf)
            gates = (jnp.dot(onehot, ce_projb,
                             preferred_element_type=jnp.float32)
                     + jnp.dot(h.astype(bf), whh1t,
                               preferred_element_type=jnp.float32))
            sif = _sigmoid(gates[:, :2 * Hc])
            g = jnp.tanh(gates[:, 2 * Hc:3 * Hc])
            so = _sigmoid(gates[:, 3 * Hc:])
            c = sif[:, Hc:] * c + sif[:, :Hc] * g
            h = so * jnp.tanh(c)

        pltpu.make_async_copy(wemb_scr.at[pl.ds(0, W)],
                              wemb_scr.at[pl.ds(0, W)], gsem).wait()
        pltpu.make_async_copy(wih2_scr, wih2_scr, wsem).wait()
        pltpu.make_async_copy(whh2_scr, whh2_scr, wsem).wait()
        pltpu.make_async_copy(wout_scr, wout_scr, wsem).wait()
        wih2t = wih2_scr[...].T.astype(bf)
        b2 = bih2_ref[...] + bhh2_ref[...]
        x2 = jnp.concatenate(
            [h.astype(bf), wemb_scr[...].astype(bf)], axis=1)
        xg2_scr[...] = (jnp.dot(x2, wih2t,
                                preferred_element_type=jnp.float32) + b2)
        whh2t = whh2_scr[...].T.astype(bf)
        woutt = wout_scr[...].T.astype(bf)

        h2 = h0_ref[...]
        c2 = c0_ref[...]
        for w in range(W):
            gates = xg2_scr[w:w + 1, :] + jnp.dot(
                h2.astype(bf), whh2t, preferred_element_type=jnp.float32)
            sif = _sigmoid(gates[:, :2 * H])
            g = jnp.tanh(gates[:, 2 * H:3 * H])
            so = _sigmoid(gates[:, 3 * H:])
            c2 = sif[:, H:] * c2 + sif[:, :H] * g
            h2 = so * jnp.tanh(c2)
            ho_scr[w:w + 1, :] = h2.astype(bf)

        tag = (jnp.dot(ho_scr[...], woutt,
                       preferred_element_type=jnp.float32) + bout_ref[...])
        m = jnp.max(tag, axis=1, keepdims=True)
        z = tag - m
        lse = jnp.log(jnp.sum(jnp.exp(z), axis=1, keepdims=True))
        out_ref[...] = z - lse

    vmem_inputs = (char_embedding,
                   w_ih1, w_hh1, b_ih1.reshape(1, -1), b_hh1.reshape(1, -1),
                   c_hx.reshape(1, Hc), c_cx.reshape(1, Hc),
                   b_ih2.reshape(1, -1), b_hh2.reshape(1, -1),
                   hx.reshape(1, H), cx.reshape(1, H),
                   b_out.reshape(1, -1))

    def _full(shape):
        nd = len(shape)
        return pl.BlockSpec(shape, lambda i, _nd=nd: (0,) * _nd)

    any_spec = pl.BlockSpec(memory_space=pl.ANY)
    out = pl.pallas_call(
        body,
        out_shape=jax.ShapeDtypeStruct((Wp, T), jnp.float32),
        grid=(1,),
        in_specs=[pl.BlockSpec(memory_space=pltpu.SMEM),
                  _full(cseq.shape),
                  any_spec, any_spec, any_spec, any_spec]
                 + [_full(x.shape) for x in vmem_inputs],
        out_specs=_full((Wp, T)),
        scratch_shapes=[pltpu.VMEM((Wp, Ew), jnp.float32),
                        pltpu.VMEM(w_ih2.shape, jnp.float32),
                        pltpu.VMEM(w_hh2.shape, jnp.float32),
                        pltpu.VMEM(w_out.shape, jnp.float32),
                        pltpu.VMEM((Wp, 4 * H), jnp.float32),
                        pltpu.VMEM((Wp, H), bf),
                        pltpu.SemaphoreType.DMA,
                        pltpu.SemaphoreType.DMA],
        compiler_params=pltpu.CompilerParams(
            dimension_semantics=("arbitrary",)),
    )(w_seq, cseq, word_embedding, w_ih2, w_hh2, w_out, *vmem_inputs)
    if Wp != W:
        out = out[:W]
    return out
